# baseline (speedup 1.0000x reference)
import functools

import jax
import jax.numpy as jnp
from jax import lax
from jax.experimental import pallas as pl
from jax.experimental.pallas import tpu as pltpu
from jax.experimental.pallas import tpu_sc as plsc

N = 100000
IN_DIM = 128
HID = 64

NC = 2
NS = 16
NW = NC * NS
SUB = 128
NSUB = 25
PER_W = SUB * NSUB
NPAD = NW * PER_W
LANES = 16
QV = HID // LANES


def _mm_body(x_ref, wt_ref, b_ref, o_ref):
    h = jnp.dot(x_ref[...], wt_ref[...], preferred_element_type=jnp.float32)
    o_ref[...] = (h + b_ref[...]) * 0.5


def _linear_half(x, Wt, b, bn=512):
    n, k = x.shape
    return pl.pallas_call(
        _mm_body,
        grid=(NPAD // bn,),
        in_specs=[
            pl.BlockSpec((bn, k), lambda i: (i, 0)),
            pl.BlockSpec((k, HID), lambda i: (0, 0)),
            pl.BlockSpec((1, HID), lambda i: (0, 0)),
        ],
        out_specs=pl.BlockSpec((bn, HID), lambda i: (i, 0)),
        out_shape=jax.ShapeDtypeStruct((NPAD, HID), jnp.float32),
    )(x, Wt, b.reshape(1, HID))


C0 = 2 * NSUB
P0 = C0 // 2
B0 = C0 * SUB


def _sc_body(h_hbm, e_hbm, out_hbm, idx_v, rows0, rows1, self01, out01,
             sem0, sem1):
    c = lax.axis_index("c")
    s = lax.axis_index("s")
    is0 = c == 0
    base = s * B0
    npairs = jnp.where(is0, P0, 0)
    blk = s * P0
    rows = (rows0, rows1)
    sems = (sem0, sem1)

    def gathers(b):
        rows_v, sem = rows[b], sems[b]
        return [
            pltpu.async_copy(
                h_hbm.at[idx_v.at[3 * b + j]],
                rows_v.at[pl.ds(j * SUB, SUB)],
                sem,
            )
            for j in range(3)
        ]

    def compute(b):
        rows_v = rows[b]
        off = b * SUB

        def node(r, c):
            for q in range(QV):
                ds = pl.ds(q * LANES, LANES)
                acc = (rows_v[r, ds] + rows_v[SUB + r, ds]
                       + rows_v[2 * SUB + r, ds] + self01[off + r, ds])
                g = acc * 0.5
                out01[off + r, ds] = jnp.where(g > 0.0, g, jnp.exp(g) - 1.0)
            return c

        lax.fori_loop(0, SUB, node, 0, unroll=2)

    def pair(p, carry):
        boff = base + 2 * p * SUB
        pltpu.sync_copy(e_hbm.at[blk + p], idx_v)
        cps0 = gathers(0)
        cps1 = gathers(1)
        pltpu.sync_copy(h_hbm.at[pl.ds(boff, 2 * SUB)], self01)
        for cp in cps0:
            cp.wait()
        compute(0)
        for cp in cps1:
            cp.wait()
        compute(1)
        pltpu.sync_copy(out01, out_hbm.at[pl.ds(boff, 2 * SUB)])
        return carry

    lax.fori_loop(0, npairs, pair, 0)


@functools.partial(
    pl.kernel,
    out_type=jax.ShapeDtypeStruct((NPAD, HID), jnp.float32),
    mesh=plsc.VectorSubcoreMesh(
        core_axis_name="c", subcore_axis_name="s", num_cores=NC, num_subcores=NS
    ),
    scratch_types=[
        pltpu.VMEM((6, SUB), jnp.int32),
        pltpu.VMEM((3 * SUB, HID), jnp.float32),
        pltpu.VMEM((3 * SUB, HID), jnp.float32),
        pltpu.VMEM((2 * SUB, HID), jnp.float32),
        pltpu.VMEM((2 * SUB, HID), jnp.float32),
        pltpu.SemaphoreType.DMA,
        pltpu.SemaphoreType.DMA,
    ],
    compiler_params=pltpu.CompilerParams(use_tc_tiling_on_sc=False),
)
def _sc_gather(h_hbm, e_hbm, out_hbm, idx_v, rows0, rows1, self01, out01,
               sem0, sem1):
    _sc_body(h_hbm, e_hbm, out_hbm, idx_v, rows0, rows1, self01, out01,
             sem0, sem1)


def _pack_edges(edge_index):
    e = jnp.zeros((3, NPAD), jnp.int32).at[:, :N].set(edge_index.T)
    return (
        e.reshape(3, NS, P0, 2, SUB)
        .transpose(1, 2, 3, 0, 4)
        .reshape(NS * P0, 6, SUB)
    )


def kernel(x, edge_index, W1, b1, W2, b2):
    xp = jnp.zeros((NPAD, IN_DIM), jnp.float32).at[:N].set(x)
    e_pairs = _pack_edges(edge_index)
    h1 = _linear_half(xp, W1.T, b1)
    g1 = _sc_gather(h1, e_pairs)
    h2 = _linear_half(g1, W2.T, b2)
    g2 = _sc_gather(h2, e_pairs)
    return g2[:N]


# speedup vs baseline: 1.1862x; 1.1862x over previous
import functools

import jax
import jax.numpy as jnp
from jax import lax
from jax.experimental import pallas as pl
from jax.experimental.pallas import tpu as pltpu
from jax.experimental.pallas import tpu_sc as plsc

N = 100000
IN_DIM = 128
HID = 64

NC = 2
NS = 16
NW = NC * NS
SUB = 128
NSUB = 25
PER_W = SUB * NSUB
NPAD = NW * PER_W
LANES = 16
QV = HID // LANES


def _mm_body(x_ref, wt_ref, b_ref, o_ref):
    h = jnp.dot(x_ref[...], wt_ref[...], preferred_element_type=jnp.float32)
    o_ref[...] = (h + b_ref[...]) * 0.5


def _linear_half(x, Wt, b, bn=512):
    n, k = x.shape
    return pl.pallas_call(
        _mm_body,
        grid=(NPAD // bn,),
        in_specs=[
            pl.BlockSpec((bn, k), lambda i: (i, 0)),
            pl.BlockSpec((k, HID), lambda i: (0, 0)),
            pl.BlockSpec((1, HID), lambda i: (0, 0)),
        ],
        out_specs=pl.BlockSpec((bn, HID), lambda i: (i, 0)),
        out_shape=jax.ShapeDtypeStruct((NPAD, HID), jnp.float32),
    )(x, Wt, b.reshape(1, HID))


C0 = 38
C1 = 2 * NSUB - C0
P0 = C0 // 2
P1 = C1 // 2
B0 = C0 * SUB
B1 = C1 * SUB
CORE0_TOTAL = NS * B0


def _sc_body(h_hbm, e_hbm, out_hbm, idx_v, rows0, rows1, self01, out01,
             sem0, sem1):
    c = lax.axis_index("c")
    s = lax.axis_index("s")
    is0 = c == 0
    base = jnp.where(is0, s * B0, CORE0_TOTAL + s * B1)
    npairs = jnp.where(is0, P0, P1)
    blk = jnp.where(is0, s * P0, NS * P0 + s * P1)
    rows = (rows0, rows1)
    sems = (sem0, sem1)

    def gathers(b):
        rows_v, sem = rows[b], sems[b]
        return [
            pltpu.async_copy(
                h_hbm.at[idx_v.at[3 * b + j]],
                rows_v.at[pl.ds(j * SUB, SUB)],
                sem,
            )
            for j in range(3)
        ]

    def compute(b):
        rows_v = rows[b]
        off = b * SUB

        def node(r, c):
            for q in range(QV):
                ds = pl.ds(q * LANES, LANES)
                acc = (rows_v[r, ds] + rows_v[SUB + r, ds]
                       + rows_v[2 * SUB + r, ds] + self01[off + r, ds])
                g = acc * 0.5
                out01[off + r, ds] = jnp.where(g > 0.0, g, jnp.exp(g) - 1.0)
            return c

        lax.fori_loop(0, SUB, node, 0, unroll=2)

    def pair(p, carry):
        boff = base + 2 * p * SUB
        pltpu.sync_copy(e_hbm.at[blk + p], idx_v)
        cps0 = gathers(0)
        cps1 = gathers(1)
        pltpu.sync_copy(h_hbm.at[pl.ds(boff, 2 * SUB)], self01)
        for cp in cps0:
            cp.wait()
        compute(0)
        for cp in cps1:
            cp.wait()
        compute(1)
        pltpu.sync_copy(out01, out_hbm.at[pl.ds(boff, 2 * SUB)])
        return carry

    lax.fori_loop(0, npairs, pair, 0)


@functools.partial(
    pl.kernel,
    out_type=jax.ShapeDtypeStruct((NPAD, HID), jnp.float32),
    mesh=plsc.VectorSubcoreMesh(
        core_axis_name="c", subcore_axis_name="s", num_cores=NC, num_subcores=NS
    ),
    scratch_types=[
        pltpu.VMEM((6, SUB), jnp.int32),
        pltpu.VMEM((3 * SUB, HID), jnp.float32),
        pltpu.VMEM((3 * SUB, HID), jnp.float32),
        pltpu.VMEM((2 * SUB, HID), jnp.float32),
        pltpu.VMEM((2 * SUB, HID), jnp.float32),
        pltpu.SemaphoreType.DMA,
        pltpu.SemaphoreType.DMA,
    ],
    compiler_params=pltpu.CompilerParams(use_tc_tiling_on_sc=False),
)
def _sc_gather(h_hbm, e_hbm, out_hbm, idx_v, rows0, rows1, self01, out01,
               sem0, sem1):
    _sc_body(h_hbm, e_hbm, out_hbm, idx_v, rows0, rows1, self01, out01,
             sem0, sem1)


def _pack_section(e, p):
    return (
        e.reshape(3, NS, p, 2, SUB)
        .transpose(1, 2, 3, 0, 4)
        .reshape(NS * p, 6, SUB)
    )


def _pack_edges(edge_index):
    e = jnp.zeros((3, NPAD), jnp.int32).at[:, :N].set(edge_index.T)
    return jnp.concatenate(
        [
            _pack_section(e[:, :CORE0_TOTAL], P0),
            _pack_section(e[:, CORE0_TOTAL:], P1),
        ],
        axis=0,
    )


def kernel(x, edge_index, W1, b1, W2, b2):
    xp = jnp.zeros((NPAD, IN_DIM), jnp.float32).at[:N].set(x)
    e_pairs = _pack_edges(edge_index)
    h1 = _linear_half(xp, W1.T, b1)
    g1 = _sc_gather(h1, e_pairs)
    h2 = _linear_half(g1, W2.T, b2)
    g2 = _sc_gather(h2, e_pairs)
    return g2[:N]
